# Initial kernel scaffold; baseline (speedup 1.0000x reference)
import jax
import jax.numpy as jnp
from jax.experimental import pallas as pl


def kernel(x):
    raise NotImplementedError("write your pallas kernel here")


# speedup vs baseline: 21.7575x; 21.7575x over previous
import functools

import jax
import jax.numpy as jnp
from jax import lax
from jax.experimental import pallas as pl
from jax.experimental.pallas import tpu as pltpu
from jax.experimental.pallas import tpu_sc as plsc

_TOP_P = 0.05
_MASK_EPSILON = 0.01
_GUMBEL_SCALE = 1.0 / (_MASK_EPSILON + 1e-06)

_LANES = 128
_NW = 32
_SCL = 16
_NB_HI = 32768
_NB_LO = 65536


def _pick_div(total, target, mult):
    t = min(target, total)
    t -= t % mult
    for d in range(t, 0, -mult):
        if total % d == 0:
            return d
    raise ValueError(f"no divisor of {total} <= {target} multiple of {mult}")


def _stats_body(x_ref, o_ref):
    j = pl.program_id(1)
    xb = x_ref[0]
    s1 = jnp.sum(xb, axis=0, keepdims=True)
    s2 = jnp.sum(xb * xb, axis=0, keepdims=True)
    rows = jnp.concatenate(
        [s1, s2, jnp.zeros((6, _LANES), jnp.float32)], axis=0)

    @pl.when(j == 0)
    def _():
        o_ref[0] = rows

    @pl.when(j != 0)
    def _():
        o_ref[0] = o_ref[0] + rows


def _acts_body(n, x_ref, g_ref, st_ref, a_ref):
    st = st_ref[0]
    s1 = jnp.sum(st[0:1, :])
    s2 = jnp.sum(st[1:2, :])
    var = (s2 - s1 * (s1 / n)) / (n - 1)
    beta = jnp.sqrt(var) * _GUMBEL_SCALE
    y = x_ref[0] + beta * g_ref[0]
    a_ref[0] = jnp.where(y > 0, y, 0.0)


def _sc_hist_hi_body(nseg, nchunks, ch, acts_ref, hist_hbm, chunk_v, hist_v):
    w = lax.axis_index("s") * 2 + lax.axis_index("c")
    base = w * nseg

    def zero_body(i, carry):
        hist_v[pl.ds(i * _SCL, _SCL)] = jnp.zeros((_SCL,), jnp.int32)
        return carry

    lax.fori_loop(0, _NB_HI // _SCL, zero_body, 0)

    ones = jnp.ones((_SCL,), jnp.int32)

    def chunk_body(c, carry):
        pltpu.sync_copy(acts_ref.at[pl.ds(base + c * ch, ch)], chunk_v)

        def vec_body(i, inner):
            bits = chunk_v[pl.ds(i * _SCL, _SCL)]
            hi = bits >> 16
            plsc.addupdate_scatter(hist_v, [hi], ones)
            return inner

        lax.fori_loop(0, ch // _SCL, vec_body, 0, unroll=4)
        return carry

    lax.fori_loop(0, nchunks, chunk_body, 0)
    pltpu.sync_copy(hist_v, hist_hbm.at[w])


def _sc_hist_lo_body(nseg, nchunks, ch, seg_per_b,
                     acts_ref, p_hbm, hist_hbm, chunk_v, p_v, hist_v):
    w = lax.axis_index("s") * 2 + lax.axis_index("c")
    base = w * nseg
    b = w // seg_per_b

    pltpu.sync_copy(p_hbm, p_v)
    bvec = jnp.full((_SCL,), b, jnp.int32)
    pvec = plsc.load_gather(p_v, [bvec])

    def zero_body(i, carry):
        hist_v[pl.ds(i * _SCL, _SCL)] = jnp.zeros((_SCL,), jnp.int32)
        return carry

    lax.fori_loop(0, _NB_LO // _SCL, zero_body, 0)

    ones = jnp.ones((_SCL,), jnp.int32)

    def chunk_body(c, carry):
        pltpu.sync_copy(acts_ref.at[pl.ds(base + c * ch, ch)], chunk_v)

        def vec_body(i, inner):
            bits = chunk_v[pl.ds(i * _SCL, _SCL)]
            hi = bits >> 16
            lo = bits & 0xFFFF
            m = hi == pvec
            plsc.addupdate_scatter(hist_v, [lo], ones, mask=m)
            return inner

        lax.fori_loop(0, ch // _SCL, vec_body, 0, unroll=4)
        return carry

    lax.fori_loop(0, nchunks, chunk_body, 0)
    pltpu.sync_copy(hist_v, hist_hbm.at[w])


def _scan_body(nb, seg, h_ref, k_ref, p_ref, ca_ref):
    nbr = nb // _LANES
    h = h_ref[0].astype(jnp.float32)
    hb = jnp.sum(h, axis=0)

    ut = (lax.broadcasted_iota(jnp.int32, (_LANES, _LANES), 0)
          >= lax.broadcasted_iota(jnp.int32, (_LANES, _LANES), 1)
          ).astype(jnp.float32)
    s_row = jnp.dot(hb, ut, precision=jax.lax.Precision.HIGHEST,
                    preferred_element_type=jnp.float32)
    t_col = s_row[:, 0:1]
    m2 = (lax.broadcasted_iota(jnp.int32, (nbr, nbr), 0)
          < lax.broadcasted_iota(jnp.int32, (nbr, nbr), 1)
          ).astype(jnp.float32)
    r_col = jnp.dot(m2, t_col, precision=jax.lax.Precision.HIGHEST,
                    preferred_element_type=jnp.float32)
    a = s_row - hb + r_col

    kf = k_ref[0, 0, 0].astype(jnp.float32)
    sel = (a < kf) & (a + hb >= kf)
    fidx = (lax.broadcasted_iota(jnp.int32, (nbr, _LANES), 0) * 128
            + lax.broadcasted_iota(jnp.int32, (nbr, _LANES), 1)
            ).astype(jnp.float32)
    p_ref[0, 0, 0] = jnp.sum(jnp.where(sel, fidx, 0.0)).astype(jnp.int32)
    ca_ref[0, 0, 0] = jnp.sum(jnp.where(sel, a, 0.0)).astype(jnp.int32)


def _mask_body(a_ref, t_ref, v_ref, m_ref):
    a = a_ref[0]
    t0 = t_ref[0, 0, 0]
    ge = a >= t0
    v_ref[0] = jnp.where(ge, a, 0.0)
    m_ref[0] = ge.astype(jnp.float32)


def kernel(x):
    B = x.shape[0]
    n = x.size // B
    k = max(1, int(_TOP_P * n))
    nr = n // _LANES

    g = jax.random.gumbel(jax.random.key(42), x.shape, dtype=x.dtype)
    xf = x.reshape(B, nr, _LANES)
    gf = g.reshape(B, nr, _LANES)

    r1 = _pick_div(nr, 4704, 8)
    g1 = nr // r1
    stats = pl.pallas_call(
        _stats_body,
        grid=(B, g1),
        in_specs=[pl.BlockSpec((1, r1, _LANES), lambda b, j: (b, j, 0))],
        out_specs=pl.BlockSpec((1, 8, _LANES), lambda b, j: (b, 0, 0)),
        out_shape=jax.ShapeDtypeStruct((B, 8, _LANES), jnp.float32),
    )(xf)

    r2 = _pick_div(nr, 2352, 8)
    g2 = nr // r2
    acts = pl.pallas_call(
        functools.partial(_acts_body, float(n)),
        grid=(B, g2),
        in_specs=[
            pl.BlockSpec((1, r2, _LANES), lambda b, j: (b, j, 0)),
            pl.BlockSpec((1, r2, _LANES), lambda b, j: (b, j, 0)),
            pl.BlockSpec((1, 8, _LANES), lambda b, j: (b, 0, 0)),
        ],
        out_specs=pl.BlockSpec((1, r2, _LANES), lambda b, j: (b, j, 0)),
        out_shape=jax.ShapeDtypeStruct((B, nr, _LANES), jnp.float32),
    )(xf, gf, stats)

    acts_flat = lax.bitcast_convert_type(acts, jnp.int32).reshape(B * n)

    seg_per_b = _NW // B
    nseg = n // seg_per_b
    ch = _pick_div(nseg, 21504, 16)
    nchunks = nseg // ch
    mesh = plsc.VectorSubcoreMesh(core_axis_name="c", subcore_axis_name="s",
                                  num_cores=2, num_subcores=16)
    hist_hi = pl.kernel(
        functools.partial(_sc_hist_hi_body, nseg, nchunks, ch),
        out_type=jax.ShapeDtypeStruct((_NW, _NB_HI), jnp.int32),
        mesh=mesh,
        scratch_types=[pltpu.VMEM((ch,), jnp.int32),
                       pltpu.VMEM((_NB_HI,), jnp.int32)],
        compiler_params=pltpu.CompilerParams(needs_layout_passes=False),
    )(acts_flat)

    kvec = jnp.full((B, 1, 1), k, jnp.int32)
    scan = lambda nb: pl.pallas_call(
        functools.partial(_scan_body, nb, seg_per_b),
        grid=(B,),
        in_specs=[
            pl.BlockSpec((1, seg_per_b, nb // _LANES, _LANES),
                         lambda b: (b, 0, 0, 0)),
            pl.BlockSpec((1, 1, 1), lambda b: (b, 0, 0), memory_space=pltpu.SMEM),
        ],
        out_specs=[
            pl.BlockSpec((1, 1, 1), lambda b: (b, 0, 0), memory_space=pltpu.SMEM),
            pl.BlockSpec((1, 1, 1), lambda b: (b, 0, 0), memory_space=pltpu.SMEM),
        ],
        out_shape=[jax.ShapeDtypeStruct((B, 1, 1), jnp.int32),
                   jax.ShapeDtypeStruct((B, 1, 1), jnp.int32)],
    )
    hist_hi4 = hist_hi.reshape(B, seg_per_b, _NB_HI // _LANES, _LANES)
    p_hi, ca_hi = scan(_NB_HI)(hist_hi4, kvec)

    p_pad = jnp.zeros((_SCL,), jnp.int32).at[:B].set(p_hi[:, 0, 0])
    hist_lo = pl.kernel(
        functools.partial(_sc_hist_lo_body, nseg, nchunks, ch, seg_per_b),
        out_type=jax.ShapeDtypeStruct((_NW, _NB_LO), jnp.int32),
        mesh=mesh,
        scratch_types=[pltpu.VMEM((ch,), jnp.int32),
                       pltpu.VMEM((_SCL,), jnp.int32),
                       pltpu.VMEM((_NB_LO,), jnp.int32)],
        compiler_params=pltpu.CompilerParams(needs_layout_passes=False),
    )(acts_flat, p_pad)

    rvec = kvec - ca_hi
    hist_lo4 = hist_lo.reshape(B, seg_per_b, _NB_LO // _LANES, _LANES)
    p_lo, _ = scan(_NB_LO)(hist_lo4, rvec)

    tbits = (p_hi[:, 0, 0] << 16) | p_lo[:, 0, 0]
    thr = lax.bitcast_convert_type(tbits, jnp.float32).reshape(B, 1, 1)

    vals, mask = pl.pallas_call(
        _mask_body,
        grid=(B, g2),
        in_specs=[
            pl.BlockSpec((1, r2, _LANES), lambda b, j: (b, j, 0)),
            pl.BlockSpec((1, 1, 1), lambda b, j: (b, 0, 0), memory_space=pltpu.SMEM),
        ],
        out_specs=[
            pl.BlockSpec((1, r2, _LANES), lambda b, j: (b, j, 0)),
            pl.BlockSpec((1, r2, _LANES), lambda b, j: (b, j, 0)),
        ],
        out_shape=[jax.ShapeDtypeStruct((B, nr, _LANES), jnp.float32),
                   jax.ShapeDtypeStruct((B, nr, _LANES), jnp.float32)],
    )(acts, thr)

    vals = vals.reshape(x.shape)
    mask = mask.reshape(B, n)
    return (vals, mask, vals, mask)
